# Initial kernel scaffold; baseline (speedup 1.0000x reference)
import jax
import jax.numpy as jnp
from jax.experimental import pallas as pl


def kernel(x, tables, W1, W2, W3):
    raise NotImplementedError("write your pallas kernel here")


# speedup vs baseline: 35.4365x; 35.4365x over previous
import functools

import numpy as np
import jax
import jax.numpy as jnp
from jax import lax
from jax.experimental import pallas as pl
from jax.experimental.pallas import tpu as pltpu
from jax.experimental.pallas import tpu_sc as plsc

LEVELS = 16
CHANNELS = 2
TABLE_SIZE = 1 << 15
BASE_RES = 16
GROWTH = 1.5
IN_COORDS = 3
HIDDEN = 64
OUT_DIM = 5
BATCH = 131072
ENC_DIM = IN_COORDS * LEVELS * CHANNELS
NPAIR = IN_COORDS * LEVELS
NWORKERS = 32
UNITS_PER_W = 3
HALF = BATCH // 2
CHUNK = 2048
NCHUNK = HALF // CHUNK
NVEC = CHUNK // 16
HASH_K = -1640531535


def _level_consts():
    scales, res = [], []
    for l in range(LEVELS):
        s = float(2.0 ** (l * np.log2(GROWTH)) * BASE_RES - 1.0)
        r = int(np.ceil(s)) + 1
        scales.append(s)
        res.append(r)
    return np.array(scales, np.float32), np.array(res, np.int32)


_SCALES, _RES = _level_consts()


def _make_encoder():
    mesh = plsc.VectorSubcoreMesh(core_axis_name="c", subcore_axis_name="s")

    @functools.partial(
        pl.kernel,
        mesh=mesh,
        out_type=jax.ShapeDtypeStruct((ENC_DIM * BATCH,), jnp.float32),
        compiler_params=pltpu.CompilerParams(needs_layout_passes=False),
        scratch_types=[
            pltpu.VMEM((TABLE_SIZE * CHANNELS,), jnp.float32),
            pltpu.VMEM((CHUNK,), jnp.float32),
            pltpu.VMEM((CHUNK,), jnp.float32),
            pltpu.VMEM((CHUNK,), jnp.float32),
            pltpu.VMEM((CHUNK,), jnp.float32),
            pltpu.VMEM((LEVELS,), jnp.float32),
            pltpu.VMEM((LEVELS,), jnp.int32),
        ],
    )
    def encode(xt, tab, scales, resa, out, table_v, xr, yr, o0, o1, sc_v, rs_v):
        pltpu.sync_copy(scales, sc_v)
        pltpu.sync_copy(resa, rs_v)
        wid = lax.axis_index("s") * 2 + lax.axis_index("c")
        for u in range(UNITS_PER_W):
            unit = wid * UNITS_PER_W + u
            pair = unit >> 1
            halfsel = unit & 1
            coord = pair >> 4
            level = pair & 15
            base = halfsel * HALF
            pltpu.sync_copy(tab.at[pl.ds(pair * (TABLE_SIZE * CHANNELS),
                                         TABLE_SIZE * CHANNELS)], table_v)
            lvl_v = jnp.full((16,), level, jnp.int32)
            scale_v = plsc.load_gather(sc_v, [lvl_v])
            res_v = plsc.load_gather(rs_v, [lvl_v])
            resm1 = res_v - 1
            is_hash = (res_v * res_v) > TABLE_SIZE
            xrow_off = coord * 2 * BATCH + base
            yrow_off = xrow_off + BATCH
            orow_off = pair * 2 * BATCH + base

            def chunk_body(ci, carry):
                off = ci * CHUNK
                pltpu.sync_copy(xt.at[pl.ds(xrow_off + off, CHUNK)], xr)
                pltpu.sync_copy(xt.at[pl.ds(yrow_off + off, CHUNK)], yr)

                def vec_body(i, carry2):
                    s0 = pl.multiple_of(i * 16, 16)
                    xv = xr[pl.ds(s0, 16)]
                    yv = yr[pl.ds(s0, 16)]
                    px = xv * scale_v + 0.5
                    py = yv * scale_v + 0.5
                    p0x = px.astype(jnp.int32)
                    p0y = py.astype(jnp.int32)
                    wx = px - p0x.astype(jnp.float32)
                    wy = py - p0y.astype(jnp.float32)
                    cx1 = jnp.minimum(p0x + 1, resm1)
                    cy1 = jnp.minimum(p0y + 1, resm1)
                    wx0 = 1.0 - wx
                    wy0 = 1.0 - wy
                    acc0 = jnp.zeros((16,), jnp.float32)
                    acc1 = jnp.zeros((16,), jnp.float32)
                    for cx, cy, w in (
                        (p0x, p0y, wx0 * wy0),
                        (p0x, cy1, wx0 * wy),
                        (cx1, p0y, wx * wy0),
                        (cx1, cy1, wx * wy),
                    ):
                        direct = cx * res_v + cy
                        hashed = (cx ^ (cy * HASH_K)) & (TABLE_SIZE - 1)
                        fi = jnp.where(is_hash, hashed, direct) * 2
                        acc0 = acc0 + w * plsc.load_gather(table_v, [fi])
                        acc1 = acc1 + w * plsc.load_gather(table_v, [fi + 1])
                    o0[pl.ds(s0, 16)] = acc0
                    o1[pl.ds(s0, 16)] = acc1
                    return carry2

                lax.fori_loop(0, NVEC, vec_body, 0)
                pltpu.sync_copy(o0, out.at[pl.ds(orow_off + off, CHUNK)])
                pltpu.sync_copy(o1, out.at[pl.ds(orow_off + BATCH + off, CHUNK)])
                return carry

            lax.fori_loop(0, NCHUNK, chunk_body, 0)

    return encode


_encode = _make_encoder()


def _mlp(enc_t, w1t, w2t, w3t):
    cb = 1024

    def body(e_ref, w1_ref, w2_ref, w3_ref, o_ref):
        h = jnp.maximum(
            lax.dot(w1_ref[...], e_ref[...], preferred_element_type=jnp.float32), 0.0)
        h = jnp.maximum(
            lax.dot(w2_ref[...], h, preferred_element_type=jnp.float32), 0.0)
        o_ref[...] = lax.dot(w3_ref[...], h, preferred_element_type=jnp.float32)

    return pl.pallas_call(
        body,
        grid=(BATCH // cb,),
        in_specs=[
            pl.BlockSpec((ENC_DIM, cb), lambda i: (0, i)),
            pl.BlockSpec((HIDDEN, ENC_DIM), lambda i: (0, 0)),
            pl.BlockSpec((HIDDEN, HIDDEN), lambda i: (0, 0)),
            pl.BlockSpec((8, HIDDEN), lambda i: (0, 0)),
        ],
        out_specs=pl.BlockSpec((8, cb), lambda i: (0, i)),
        out_shape=jax.ShapeDtypeStruct((8, BATCH), jnp.float32),
    )(enc_t, w1t, w2t, w3t)


def kernel(x, tables, W1, W2, W3):
    xt = x.T.reshape(-1)
    tab = tables.reshape(-1)
    enc_flat = _encode(xt, tab, jnp.asarray(_SCALES), jnp.asarray(_RES))
    enc_t = enc_flat.reshape(ENC_DIM, BATCH)
    w1t = W1.T
    w2t = W2.T
    w3t = jnp.pad(W3, ((0, 0), (0, 3))).T
    out_t = _mlp(enc_t, w1t, w2t, w3t)
    out = out_t[:OUT_DIM].T
    return (out[:, : OUT_DIM - 1], out[:, OUT_DIM - 1:])
